# baseline (speedup 1.0000x reference)
import jax
import jax.numpy as jnp
from jax.experimental import pallas as pl
from jax.experimental.pallas import tpu as pltpu

_BM = 512
_BK = 1024
_NB = 4096 // _BM
_NK = 4096 // _BK
_S1 = _NB * _NK


def _fused_body(e_ref, tar_ref, src_ref, o_ref, t_ref):
    i = pl.program_id(0)

    @pl.when(i < _S1)
    def _stage1():
        m = jax.lax.rem(i, _NB)
        k = jax.lax.div(i, _NB)
        part = jnp.dot(tar_ref[...], e_ref[...],
                       preferred_element_type=jnp.float32,
                       precision=jax.lax.Precision.DEFAULT)
        rows = pl.ds(m * _BM, _BM)

        @pl.when(k == 0)
        def _init():
            t_ref[rows, :] = part

        @pl.when(k > 0)
        def _acc():
            t_ref[rows, :] += part

    @pl.when(i >= _S1)
    def _stage2():
        o_ref[...] = jnp.dot(src_ref[...], t_ref[...],
                             preferred_element_type=jnp.float32,
                             precision=jax.lax.Precision.DEFAULT)


def kernel(poi_embs, hg_poi_src, hg_poi_tar):
    n, kdim = hg_poi_src.shape
    _, d = poi_embs.shape

    def tar_idx(i):
        in1 = i < _S1
        m = jnp.where(in1, jax.lax.rem(i, _NB), _NB - 1)
        k = jnp.where(in1, jax.lax.div(i, _NB), _NK - 1)
        return (m, k)

    def e_idx(i):
        return (jnp.minimum(jax.lax.div(i, _NB), _NK - 1), 0)

    return pl.pallas_call(
        _fused_body,
        grid=(_S1 + _NB,),
        in_specs=[
            pl.BlockSpec((_BK, d), e_idx),
            pl.BlockSpec((_BM, _BK), tar_idx),
            pl.BlockSpec((_BM, kdim), lambda i: (jnp.maximum(i - _S1, 0), 0)),
        ],
        out_specs=pl.BlockSpec((_BM, d), lambda i: (jnp.maximum(i - _S1, 0), 0)),
        out_shape=jax.ShapeDtypeStruct((n, d), jnp.float32),
        scratch_shapes=[pltpu.VMEM((kdim, d), jnp.float32)],
        compiler_params=pltpu.CompilerParams(
            dimension_semantics=("arbitrary",),
            vmem_limit_bytes=66060288,
        ),
    )(poi_embs, hg_poi_tar, hg_poi_src)


# speedup vs baseline: 1.1102x; 1.1102x over previous
import jax
import jax.numpy as jnp
from jax.experimental import pallas as pl
from jax.experimental.pallas import tpu as pltpu

_BM = 512
_NB = 4096 // _BM


def _fused_body(e_ref, tar_ref, src_ref, o_ref, t_ref):
    i = pl.program_id(0)

    @pl.when(i < _NB)
    def _stage1():
        acc = jnp.dot(tar_ref[...], e_ref[...],
                      preferred_element_type=jnp.float32,
                      precision=jax.lax.Precision.DEFAULT)
        t_ref[pl.ds(i * _BM, _BM), :] = acc.astype(jnp.bfloat16)

    @pl.when(i >= _NB)
    def _stage2():
        o_ref[...] = jnp.dot(src_ref[...], t_ref[...],
                             preferred_element_type=jnp.float32,
                             precision=jax.lax.Precision.DEFAULT)


def kernel(poi_embs, hg_poi_src, hg_poi_tar):
    n, k = hg_poi_src.shape
    _, d = poi_embs.shape
    return pl.pallas_call(
        _fused_body,
        grid=(2 * _NB,),
        in_specs=[
            pl.BlockSpec((k, d), lambda i: (0, 0)),
            pl.BlockSpec((_BM, k), lambda i: (jnp.minimum(i, _NB - 1), 0)),
            pl.BlockSpec((_BM, k), lambda i: (jnp.maximum(i - _NB, 0), 0)),
        ],
        out_specs=pl.BlockSpec((_BM, d), lambda i: (jnp.maximum(i - _NB, 0), 0)),
        out_shape=jax.ShapeDtypeStruct((n, d), jnp.float32),
        scratch_shapes=[pltpu.VMEM((k, d), jnp.bfloat16)],
        compiler_params=pltpu.CompilerParams(
            dimension_semantics=("arbitrary",),
            vmem_limit_bytes=66060288,
        ),
    )(poi_embs, hg_poi_tar, hg_poi_src)
